# baseline (speedup 1.0000x reference)
import jax
import jax.numpy as jnp
from jax.experimental import pallas as pl
from jax.experimental.pallas import tpu as pltpu

N = 10000
BR = 320
GRID = (N + BR - 1) // BR
NPAD = GRID * BR
BR2 = 1280
GRID2 = NPAD // BR2
GROUP = BR2 // BR

_QS = 254.0


def _chunk(k):
    lo = k * BR2
    hi = min((k + 1) * BR2, N)
    return lo, hi


def _pass1_kernel(adj_ref, x_ref, h1_ref, q_ref, h2p_ref, acc_ref, p_ref):
    i = pl.program_id(0)
    g = i // GROUP
    a = adj_ref[...]
    ab = a.astype(jnp.bfloat16)
    hb = jnp.dot(ab, x_ref[...],
                 preferred_element_type=jnp.float32).astype(jnp.bfloat16)
    h1_ref[...] = hb
    acc_ref[pl.ds(i * BR, BR), :] = hb
    q_ref[...] = (a * _QS + 0.5).astype(jnp.uint8)

    p_ref[...] = jnp.zeros_like(p_ref)
    for s in range(GRID2 - 1):
        lo, hi = _chunk(s)

        @pl.when(s < g)
        def _(lo=lo, hi=hi):
            p_ref[...] += jnp.dot(ab[:, lo:hi], acc_ref[lo:hi, :],
                                  preferred_element_type=jnp.float32)

    h2p_ref[...] = p_ref[...].astype(jnp.bfloat16)


def _pass2_kernel(q_ref, h_ref, h2p_ref, W1_ref, b1_ref, W2_ref, b2_ref,
                  o_ref, h2s_ref):
    g = pl.program_id(0)
    qf = q_ref[...]
    hf = h_ref[...]
    h2s_ref[...] = h2p_ref[...].astype(jnp.float32) * _QS
    for k in range(GRID2):
        lo, hi = _chunk(k)

        @pl.when(k >= g)
        def _(lo=lo, hi=hi):
            h2s_ref[...] += jnp.dot(qf[:, lo:hi].astype(jnp.bfloat16),
                                    hf[lo:hi, :],
                                    preferred_element_type=jnp.float32)

    h2 = h2s_ref[...] * (1.0 / _QS)
    h = jnp.dot(h2, W1_ref[...], preferred_element_type=jnp.float32) + b1_ref[...]
    h = jnp.maximum(h, 0.0)
    z = jnp.dot(h, W2_ref[...], preferred_element_type=jnp.float32) + b2_ref[...]
    m = jnp.max(z, axis=1, keepdims=True)
    zs = z - m
    lse = jnp.log(jnp.sum(jnp.exp(zs), axis=1, keepdims=True))
    o_ref[...] = zs - lse


def kernel(x, adj, W1, b1, W2, b2):
    nfeat = x.shape[1]
    nclass = W2.shape[1]

    row_spec = lambda c: pl.BlockSpec((BR, c), lambda i: (i, 0))
    full = lambda shape: pl.BlockSpec(shape, lambda i: (0, 0))

    h1, q, h2p = pl.pallas_call(
        _pass1_kernel,
        grid=(GRID,),
        in_specs=[row_spec(N), full((N, nfeat))],
        out_specs=[row_spec(nfeat), row_spec(N), row_spec(nfeat)],
        out_shape=[
            jax.ShapeDtypeStruct((N, nfeat), jnp.bfloat16),
            jax.ShapeDtypeStruct((NPAD, N), jnp.uint8),
            jax.ShapeDtypeStruct((NPAD, nfeat), jnp.bfloat16),
        ],
        scratch_shapes=[
            pltpu.VMEM((NPAD, nfeat), jnp.bfloat16),
            pltpu.VMEM((BR, nfeat), jnp.float32),
        ],
    )(adj, x.astype(jnp.bfloat16))

    b1r = b1.reshape(1, -1)
    b2r = b2.reshape(1, -1)
    row_spec2 = lambda c: pl.BlockSpec((BR2, c), lambda i: (i, 0))
    out = pl.pallas_call(
        _pass2_kernel,
        grid=(GRID2,),
        in_specs=[
            row_spec2(N),
            full((N, nfeat)),
            row_spec2(nfeat),
            full(W1.shape),
            full(b1r.shape),
            full(W2.shape),
            full(b2r.shape),
        ],
        out_specs=row_spec2(nclass),
        out_shape=jax.ShapeDtypeStruct((N, nclass), jnp.float32),
        scratch_shapes=[pltpu.VMEM((BR2, nfeat), jnp.float32)],
    )(q, h1, h2p, W1, b1r, W2, b2r)
    return out


# speedup vs baseline: 1.0470x; 1.0470x over previous
import jax
import jax.numpy as jnp
from jax.experimental import pallas as pl
from jax.experimental.pallas import tpu as pltpu

N = 10000
BR = 320
GRID = (N + BR - 1) // BR
NPAD = GRID * BR
BR2 = 1280
GRID2 = NPAD // BR2
GROUP = BR2 // BR

_QS = 254.0


def _pass1_kernel(adj_ref, x_ref, h1_ref, q_ref, h2p_ref, acc_ref):
    i = pl.program_id(0)
    g = i // GROUP
    a = adj_ref[...]
    ab = a.astype(jnp.bfloat16)
    hb = jnp.dot(ab, x_ref[...],
                 preferred_element_type=jnp.float32).astype(jnp.bfloat16)
    h1_ref[...] = hb
    acc_ref[pl.ds(i * BR, BR), :] = hb
    q_ref[...] = (a * _QS + 0.5).astype(jnp.uint8)

    @pl.when(g == 0)
    def _():
        h2p_ref[...] = jnp.zeros_like(h2p_ref)

    for G in range(1, GRID2):
        w = G * BR2

        @pl.when(g == G)
        def _(w=w):
            h2p_ref[...] = jnp.dot(
                ab[:, :w], acc_ref[:w, :],
                preferred_element_type=jnp.float32).astype(jnp.bfloat16)


def _pass2_kernel(q_ref, h_ref, h2p_ref, W1_ref, b1_ref, W2_ref, b2_ref,
                  o_ref, h2s_ref):
    g = pl.program_id(0)
    qf = q_ref[...]
    hf = h_ref[...]
    for G in range(GRID2):
        lo = G * BR2

        @pl.when(g == G)
        def _(lo=lo):
            qm = jnp.dot(qf[:, lo:].astype(jnp.bfloat16), hf[lo:, :],
                         preferred_element_type=jnp.float32)
            h2s_ref[...] = qm + h2p_ref[...].astype(jnp.float32) * _QS

    h2 = h2s_ref[...] * (1.0 / _QS)
    h = jnp.dot(h2, W1_ref[...], preferred_element_type=jnp.float32) + b1_ref[...]
    h = jnp.maximum(h, 0.0)
    z = jnp.dot(h, W2_ref[...], preferred_element_type=jnp.float32) + b2_ref[...]
    m = jnp.max(z, axis=1, keepdims=True)
    zs = z - m
    lse = jnp.log(jnp.sum(jnp.exp(zs), axis=1, keepdims=True))
    o_ref[...] = zs - lse


def kernel(x, adj, W1, b1, W2, b2):
    nfeat = x.shape[1]
    nclass = W2.shape[1]

    row_spec = lambda c: pl.BlockSpec((BR, c), lambda i: (i, 0))
    full = lambda shape: pl.BlockSpec(shape, lambda i: (0, 0))

    h1, q, h2p = pl.pallas_call(
        _pass1_kernel,
        grid=(GRID,),
        in_specs=[row_spec(N), full((N, nfeat))],
        out_specs=[row_spec(nfeat), row_spec(N), row_spec(nfeat)],
        out_shape=[
            jax.ShapeDtypeStruct((N, nfeat), jnp.bfloat16),
            jax.ShapeDtypeStruct((NPAD, N), jnp.uint8),
            jax.ShapeDtypeStruct((NPAD, nfeat), jnp.bfloat16),
        ],
        scratch_shapes=[pltpu.VMEM((NPAD, nfeat), jnp.bfloat16)],
    )(adj, x.astype(jnp.bfloat16))

    b1r = b1.reshape(1, -1)
    b2r = b2.reshape(1, -1)
    row_spec2 = lambda c: pl.BlockSpec((BR2, c), lambda i: (i, 0))
    out = pl.pallas_call(
        _pass2_kernel,
        grid=(GRID2,),
        in_specs=[
            row_spec2(N),
            full((N, nfeat)),
            row_spec2(nfeat),
            full(W1.shape),
            full(b1r.shape),
            full(W2.shape),
            full(b2r.shape),
        ],
        out_specs=row_spec2(nclass),
        out_shape=jax.ShapeDtypeStruct((N, nclass), jnp.float32),
        scratch_shapes=[pltpu.VMEM((BR2, nfeat), jnp.float32)],
    )(q, h1, h2p, W1, b1r, W2, b2r)
    return out


# speedup vs baseline: 1.0634x; 1.0157x over previous
import jax
import jax.numpy as jnp
from jax.experimental import pallas as pl
from jax.experimental.pallas import tpu as pltpu

N = 10000
BR = 320
GRID = (N + BR - 1) // BR
NPAD = GRID * BR
BR2 = 1024
GRID2 = NPAD // BR2

SPLIT = NPAD // 2
HALF1 = GRID // 2
HALF2 = GRID2 // 2

_QS = 254.0


def _pass1_kernel(adj_ref, x_ref, h1_ref, q_ref, h2p_ref, acc_ref):
    i = pl.program_id(0)
    a = adj_ref[...]
    ab = a.astype(jnp.bfloat16)
    hb = jnp.dot(ab, x_ref[...],
                 preferred_element_type=jnp.float32).astype(jnp.bfloat16)
    h1_ref[...] = hb
    acc_ref[pl.ds(i * BR, BR), :] = hb
    q_ref[...] = (a * _QS + 0.5).astype(jnp.uint8)

    @pl.when(i < HALF1)
    def _():
        h2p_ref[...] = jnp.zeros_like(h2p_ref)

    @pl.when(i >= HALF1)
    def _():
        h2p_ref[...] = jnp.dot(
            ab[:, :SPLIT], acc_ref[:SPLIT, :],
            preferred_element_type=jnp.float32).astype(jnp.bfloat16)


def _pass2_kernel(q_ref, h_ref, h2p_ref, W1_ref, b1_ref, W2_ref, b2_ref,
                  o_ref, h2s_ref):
    i = pl.program_id(0)

    @pl.when(i < HALF2)
    def _():
        h2s_ref[...] = jnp.dot(q_ref[...].astype(jnp.bfloat16), h_ref[...],
                               preferred_element_type=jnp.float32) * (1.0 / _QS)

    @pl.when(i >= HALF2)
    def _():
        qm = jnp.dot(q_ref[...][:, SPLIT:].astype(jnp.bfloat16),
                     h_ref[...][SPLIT:, :],
                     preferred_element_type=jnp.float32) * (1.0 / _QS)
        h2s_ref[...] = qm + h2p_ref[...].astype(jnp.float32)

    h2 = h2s_ref[...]
    h = jnp.dot(h2, W1_ref[...], preferred_element_type=jnp.float32) + b1_ref[...]
    h = jnp.maximum(h, 0.0)
    z = jnp.dot(h, W2_ref[...], preferred_element_type=jnp.float32) + b2_ref[...]
    m = jnp.max(z, axis=1, keepdims=True)
    zs = z - m
    lse = jnp.log(jnp.sum(jnp.exp(zs), axis=1, keepdims=True))
    o_ref[...] = zs - lse


def kernel(x, adj, W1, b1, W2, b2):
    nfeat = x.shape[1]
    nclass = W2.shape[1]

    row_spec = lambda c: pl.BlockSpec((BR, c), lambda i: (i, 0))
    full = lambda shape: pl.BlockSpec(shape, lambda i: (0, 0))

    h1, q, h2p = pl.pallas_call(
        _pass1_kernel,
        grid=(GRID,),
        in_specs=[row_spec(N), full((N, nfeat))],
        out_specs=[row_spec(nfeat), row_spec(N), row_spec(nfeat)],
        out_shape=[
            jax.ShapeDtypeStruct((N, nfeat), jnp.bfloat16),
            jax.ShapeDtypeStruct((NPAD, N), jnp.uint8),
            jax.ShapeDtypeStruct((NPAD, nfeat), jnp.bfloat16),
        ],
        scratch_shapes=[pltpu.VMEM((NPAD, nfeat), jnp.bfloat16)],
    )(adj, x.astype(jnp.bfloat16))

    b1r = b1.reshape(1, -1)
    b2r = b2.reshape(1, -1)
    row_spec2 = lambda c: pl.BlockSpec((BR2, c), lambda i: (i, 0))
    out = pl.pallas_call(
        _pass2_kernel,
        grid=(GRID2,),
        in_specs=[
            row_spec2(N),
            full((N, nfeat)),
            row_spec2(nfeat),
            full(W1.shape),
            full(b1r.shape),
            full(W2.shape),
            full(b2r.shape),
        ],
        out_specs=row_spec2(nclass),
        out_shape=jax.ShapeDtypeStruct((N, nclass), jnp.float32),
        scratch_shapes=[pltpu.VMEM((BR2, nfeat), jnp.float32)],
    )(q, h1, h2p, W1, b1r, W2, b2r)
    return out
